# Initial kernel scaffold; baseline (speedup 1.0000x reference)
import jax
import jax.numpy as jnp
from jax.experimental import pallas as pl


def kernel(x, edge_index, batch, W0, b0, g0, be0, W1, b1, g1, be1, W2, b2, g2, be2, Wm1, bm1, Wm2, bm2):
    raise NotImplementedError("write your pallas kernel here")


# speedup vs baseline: 10.5059x; 10.5059x over previous
import functools

import jax
import jax.numpy as jnp
from jax import lax
from jax.experimental import pallas as pl
from jax.experimental.pallas import tpu as pltpu
from jax.experimental.pallas import tpu_sc as plsc

NN = 10000
EE = 320000
FI = 128
DM = 20
NP = 10112
EP = 327680
NW = 32
CH = 128
NCHUNK = EP // NW // CH
RPS = NP // 16

_mesh = plsc.VectorSubcoreMesh(core_axis_name="c", subcore_axis_name="s")
_f32 = jnp.float32
_sc_params = pltpu.CompilerParams(use_tc_tiling_on_sc=False)


@functools.partial(
    pl.kernel, mesh=_mesh, compiler_params=_sc_params,
    out_type=jax.ShapeDtypeStruct((2, NP, 8), _f32),
    scratch_types=[
        pltpu.VMEM((NCHUNK, CH), jnp.int32),
        pltpu.VMEM((CH, 8), _f32),
        pltpu.VMEM_SHARED((NP, 8), _f32),
    ],
)
def _sc_deg(dstm, ones_hbm, zeros_hbm, out, didx, onesv, acc):
    cid = lax.axis_index("c")
    sid = lax.axis_index("s")
    wid = sid * 2 + cid
    pltpu.sync_copy(dstm.at[wid], didx)
    pltpu.sync_copy(ones_hbm, onesv)
    pltpu.sync_copy(zeros_hbm.at[pl.ds(sid * RPS, RPS)],
                    acc.at[pl.ds(sid * RPS, RPS)])
    plsc.subcore_barrier()

    def body(j, carry):
        pltpu.sync_copy(onesv, acc.at[didx.at[j]], add=True)
        return carry

    lax.fori_loop(0, NCHUNK, body, 0)
    plsc.subcore_barrier()
    pltpu.sync_copy(acc.at[pl.ds(sid * RPS, RPS)],
                    out.at[cid, pl.ds(sid * RPS, RPS)])


@functools.partial(
    pl.kernel, mesh=_mesh, compiler_params=_sc_params,
    out_type=jax.ShapeDtypeStruct((2, NP, DM), _f32),
    scratch_types=[
        pltpu.VMEM((NCHUNK, CH), jnp.int32),
        pltpu.VMEM((NCHUNK, CH), jnp.int32),
        pltpu.VMEM((CH, DM), _f32),
        pltpu.VMEM_SHARED((NP, DM), _f32),
        pltpu.SemaphoreType.DMA,
    ],
)
def _sc_scatter(t_hbm, srcm, dstm, zeros_hbm, out, sidx, didx, rbuf, acc, sem):
    cid = lax.axis_index("c")
    sid = lax.axis_index("s")
    wid = sid * 2 + cid
    pltpu.sync_copy(srcm.at[wid], sidx)
    pltpu.sync_copy(dstm.at[wid], didx)
    pltpu.sync_copy(zeros_hbm.at[pl.ds(sid * RPS, RPS)],
                    acc.at[pl.ds(sid * RPS, RPS)])
    plsc.subcore_barrier()

    def body(j, carry):
        pltpu.async_copy(t_hbm.at[sidx.at[j]], rbuf, sem).wait()
        pltpu.sync_copy(rbuf, acc.at[didx.at[j]], add=True)
        return carry

    lax.fori_loop(0, NCHUNK, body, 0)
    plsc.subcore_barrier()
    pltpu.sync_copy(acc.at[pl.ds(sid * RPS, RPS)],
                    out.at[cid, pl.ds(sid * RPS, RPS)])


@functools.partial(
    pl.kernel, mesh=_mesh, compiler_params=_sc_params,
    out_type=(jax.ShapeDtypeStruct((EP, DM), _f32),
              jax.ShapeDtypeStruct((EP, DM), _f32)),
    scratch_types=[
        pltpu.VMEM((NCHUNK, CH), jnp.int32),
        pltpu.VMEM((NCHUNK, CH), jnp.int32),
        pltpu.VMEM((CH, DM), _f32),
        pltpu.VMEM((CH, DM), _f32),
        pltpu.SemaphoreType.DMA,
    ],
)
def _sc_edge(h_hbm, srcm, dstm, outr, outc, sidx, didx, rbuf, cbuf, sem):
    cid = lax.axis_index("c")
    sid = lax.axis_index("s")
    wid = sid * 2 + cid
    pltpu.sync_copy(srcm.at[wid], sidx)
    pltpu.sync_copy(dstm.at[wid], didx)

    def body(j, carry):
        base = wid * (NCHUNK * CH) + j * CH
        pltpu.async_copy(h_hbm.at[sidx.at[j]], rbuf, sem).wait()
        pltpu.sync_copy(rbuf, outr.at[pl.ds(base, CH)])
        pltpu.async_copy(h_hbm.at[didx.at[j]], cbuf, sem).wait()
        pltpu.sync_copy(cbuf, outc.at[pl.ds(base, CH)])
        return carry

    lax.fori_loop(0, NCHUNK, body, 0)


def _dis_from(degp):
    deg = degp[0, :, 0:1] + degp[1, :, 0:1] + 1.0
    return 1.0 / jnp.sqrt(deg)


def _tc_a_body(x_ref, w0_ref, degp_ref, t0_ref):
    dis = _dis_from(degp_ref[...])
    h = jnp.dot(x_ref[...], w0_ref[...], preferred_element_type=_f32)
    t0_ref[...] = dis * h


def _tc_a(xp, w0, degp):
    return pl.pallas_call(
        _tc_a_body,
        out_shape=jax.ShapeDtypeStruct((NP, DM), _f32),
    )(xp, w0, degp)


def _bn_relu(conv, g, be):
    mask = lax.broadcasted_iota(jnp.int32, (NP, 1), 0) < NN
    cm = jnp.where(mask, conv, 0.0)
    m = jnp.sum(cm, axis=0, keepdims=True) / NN
    d = conv - m
    v = jnp.sum(jnp.where(mask, d * d, 0.0), axis=0, keepdims=True) / NN
    return jax.nn.relu(g * d / jnp.sqrt(v + 1e-5) + be)


def _tc_mid_body(sp_ref, tprev_ref, degp_ref, b_ref, g_ref, be_ref, wn_ref,
                 out_ref):
    dis = _dis_from(degp_ref[...])
    conv = dis * (sp_ref[0] + sp_ref[1] + tprev_ref[...]) + b_ref[...]
    hbn = _bn_relu(conv, g_ref[...], be_ref[...])
    out_ref[...] = dis * jnp.dot(hbn, wn_ref[...], preferred_element_type=_f32)


def _tc_mid(sp, tprev, degp, b, g, be, wn):
    return pl.pallas_call(
        _tc_mid_body,
        out_shape=jax.ShapeDtypeStruct((NP, DM), _f32),
    )(sp, tprev, degp, b.reshape(1, DM), g.reshape(1, DM), be.reshape(1, DM),
      wn)


def _tc_last_body(sp_ref, tprev_ref, degp_ref, b_ref, g_ref, be_ref, out_ref):
    dis = _dis_from(degp_ref[...])
    conv = dis * (sp_ref[0] + sp_ref[1] + tprev_ref[...]) + b_ref[...]
    out_ref[...] = _bn_relu(conv, g_ref[...], be_ref[...])


def _tc_last(sp, tprev, degp, b, g, be):
    return pl.pallas_call(
        _tc_last_body,
        out_shape=jax.ShapeDtypeStruct((NP, DM), _f32),
    )(sp, tprev, degp, b.reshape(1, DM), g.reshape(1, DM), be.reshape(1, DM))


_BE = 2048


def _tc_mlp_body(r_ref, c_ref, wa_ref, wb_ref, bm1_ref, wm2_ref, bm2_ref,
                 o_ref):
    r = r_ref[...]
    c = c_ref[...]
    mn = jnp.minimum(r, c)
    mx = jnp.maximum(r, c)
    z = (jnp.dot(mn, wa_ref[...], preferred_element_type=_f32)
         + jnp.dot(mx, wb_ref[...], preferred_element_type=_f32)
         + bm1_ref[...])
    o_ref[...] = (jnp.dot(jax.nn.relu(z), wm2_ref[...],
                          preferred_element_type=_f32) + bm2_ref[...])


def _tc_mlp(r, c, wm1, bm1, wm2, bm2):
    grid = (EP // _BE,)
    full = lambda s: pl.BlockSpec(s, lambda i: (0, 0))
    return pl.pallas_call(
        _tc_mlp_body,
        grid=grid,
        in_specs=[
            pl.BlockSpec((_BE, DM), lambda i: (i, 0)),
            pl.BlockSpec((_BE, DM), lambda i: (i, 0)),
            full((DM, 64)), full((DM, 64)), full((1, 64)),
            full((64, 1)), full((1, 1)),
        ],
        out_specs=pl.BlockSpec((_BE, 1), lambda i: (i, 0)),
        out_shape=jax.ShapeDtypeStruct((EP, 1), _f32),
    )(r, c, wm1[:DM], wm1[DM:], bm1.reshape(1, 64), wm2, bm2.reshape(1, 1))


def kernel(x, edge_index, batch, W0, b0, g0, be0, W1, b1, g1, be1,
           W2, b2, g2, be2, Wm1, bm1, Wm2, bm2):
    src = edge_index[0]
    dst = edge_index[1]
    pad = EP - EE
    padidx = NN + (jnp.arange(pad, dtype=jnp.int32) % 16)
    srcm = jnp.concatenate([src, padidx]).reshape(NW, NCHUNK, CH)
    dstm = jnp.concatenate([dst, padidx]).reshape(NW, NCHUNK, CH)

    zeros20 = jnp.zeros((NP, DM), _f32)
    zeros8 = jnp.zeros((NP, 8), _f32)
    ones8 = jnp.ones((CH, 8), _f32)
    xp = jnp.pad(x, ((0, NP - NN), (0, 0)))

    degp = _sc_deg(dstm, ones8, zeros8)
    t0 = _tc_a(xp, W0, degp)
    sp = _sc_scatter(t0, srcm, dstm, zeros20)
    t1 = _tc_mid(sp, t0, degp, b0, g0, be0, W1)
    sp = _sc_scatter(t1, srcm, dstm, zeros20)
    t2 = _tc_mid(sp, t1, degp, b1, g1, be1, W2)
    sp = _sc_scatter(t2, srcm, dstm, zeros20)
    h3 = _tc_last(sp, t2, degp, b2, g2, be2)
    r, c = _sc_edge(h3, srcm, dstm)
    out = _tc_mlp(r, c, Wm1, bm1, Wm2, bm2)
    return out[:EE]


# speedup vs baseline: 10.5099x; 1.0004x over previous
import functools

import jax
import jax.numpy as jnp
from jax import lax
from jax.experimental import pallas as pl
from jax.experimental.pallas import tpu as pltpu
from jax.experimental.pallas import tpu_sc as plsc

NN = 10000
EE = 320000
FI = 128
DM = 20
NP = 10112
EP = 327680
NW = 32
CH = 128
NCHUNK = EP // NW // CH
RPS = NP // 16

_mesh = plsc.VectorSubcoreMesh(core_axis_name="c", subcore_axis_name="s")
_f32 = jnp.float32
_sc_params = pltpu.CompilerParams(use_tc_tiling_on_sc=False)


@functools.partial(
    pl.kernel, mesh=_mesh, compiler_params=_sc_params,
    out_type=jax.ShapeDtypeStruct((2, NP, 8), _f32),
    scratch_types=[
        pltpu.VMEM((NCHUNK, CH), jnp.int32),
        pltpu.VMEM((CH, 8), _f32),
        pltpu.VMEM_SHARED((NP, 8), _f32),
    ],
)
def _sc_deg(dstm, ones_hbm, zeros_hbm, out, didx, onesv, acc):
    cid = lax.axis_index("c")
    sid = lax.axis_index("s")
    wid = sid * 2 + cid
    pltpu.sync_copy(dstm.at[wid], didx)
    pltpu.sync_copy(ones_hbm, onesv)
    pltpu.sync_copy(zeros_hbm.at[pl.ds(sid * RPS, RPS)],
                    acc.at[pl.ds(sid * RPS, RPS)])
    plsc.subcore_barrier()

    def body(j, carry):
        pltpu.sync_copy(onesv, acc.at[didx.at[j]], add=True)
        return carry

    lax.fori_loop(0, NCHUNK, body, 0)
    plsc.subcore_barrier()
    pltpu.sync_copy(acc.at[pl.ds(sid * RPS, RPS)],
                    out.at[cid, pl.ds(sid * RPS, RPS)])


@functools.partial(
    pl.kernel, mesh=_mesh, compiler_params=_sc_params,
    out_type=jax.ShapeDtypeStruct((2, NP, DM), _f32),
    scratch_types=[
        pltpu.VMEM((NCHUNK, CH), jnp.int32),
        pltpu.VMEM((NCHUNK, CH), jnp.int32),
        pltpu.VMEM((CH, DM), _f32),
        pltpu.VMEM_SHARED((NP, DM), _f32),
        pltpu.SemaphoreType.DMA,
    ],
)
def _sc_scatter(t_hbm, srcm, dstm, zeros_hbm, out, sidx, didx, rbuf0,
                acc, sem0):
    cid = lax.axis_index("c")
    sid = lax.axis_index("s")
    wid = sid * 2 + cid
    pltpu.sync_copy(srcm.at[wid], sidx)
    pltpu.sync_copy(dstm.at[wid], didx)
    pltpu.sync_copy(zeros_hbm.at[pl.ds(sid * RPS, RPS)],
                    acc.at[pl.ds(sid * RPS, RPS)])
    plsc.subcore_barrier()

    def body(j, carry):
        pltpu.async_copy(t_hbm.at[sidx.at[j]], rbuf0, sem0).wait()
        pltpu.sync_copy(rbuf0, acc.at[didx.at[j]], add=True)
        return carry

    lax.fori_loop(0, NCHUNK, body, 0)
    plsc.subcore_barrier()
    pltpu.sync_copy(acc.at[pl.ds(sid * RPS, RPS)],
                    out.at[cid, pl.ds(sid * RPS, RPS)])


@functools.partial(
    pl.kernel, mesh=_mesh, compiler_params=_sc_params,
    out_type=(jax.ShapeDtypeStruct((EP, DM), _f32),
              jax.ShapeDtypeStruct((EP, DM), _f32)),
    scratch_types=[
        pltpu.VMEM((NCHUNK, CH), jnp.int32),
        pltpu.VMEM((NCHUNK, CH), jnp.int32),
        pltpu.VMEM((CH, DM), _f32),
        pltpu.VMEM((CH, DM), _f32),
        pltpu.SemaphoreType.DMA,
        pltpu.SemaphoreType.DMA,
    ],
)
def _sc_edge(h_hbm, srcm, dstm, outr, outc, sidx, didx, rbuf0, cbuf0,
             sr0, sc0):
    cid = lax.axis_index("c")
    sid = lax.axis_index("s")
    wid = sid * 2 + cid
    pltpu.sync_copy(srcm.at[wid], sidx)
    pltpu.sync_copy(dstm.at[wid], didx)

    def body(j, carry):
        base = wid * (NCHUNK * CH) + j * CH
        pltpu.async_copy(h_hbm.at[sidx.at[j]], rbuf0, sr0).wait()
        pltpu.sync_copy(rbuf0, outr.at[pl.ds(base, CH)])
        pltpu.async_copy(h_hbm.at[didx.at[j]], cbuf0, sc0).wait()
        pltpu.sync_copy(cbuf0, outc.at[pl.ds(base, CH)])
        return carry

    lax.fori_loop(0, NCHUNK, body, 0)


def _dis_from(degp):
    deg = degp[0, :, 0:1] + degp[1, :, 0:1] + 1.0
    return 1.0 / jnp.sqrt(deg)


def _tc_a_body(x_ref, w0_ref, degp_ref, t0_ref):
    dis = _dis_from(degp_ref[...])
    h = jnp.dot(x_ref[...], w0_ref[...], preferred_element_type=_f32)
    t0_ref[...] = dis * h


def _tc_a(xp, w0, degp):
    return pl.pallas_call(
        _tc_a_body,
        out_shape=jax.ShapeDtypeStruct((NP, DM), _f32),
    )(xp, w0, degp)


def _bn_relu(conv, g, be):
    mask = lax.broadcasted_iota(jnp.int32, (NP, 1), 0) < NN
    cm = jnp.where(mask, conv, 0.0)
    m = jnp.sum(cm, axis=0, keepdims=True) / NN
    d = conv - m
    v = jnp.sum(jnp.where(mask, d * d, 0.0), axis=0, keepdims=True) / NN
    return jax.nn.relu(g * d / jnp.sqrt(v + 1e-5) + be)


def _tc_mid_body(sp_ref, tprev_ref, degp_ref, b_ref, g_ref, be_ref, wn_ref,
                 out_ref):
    dis = _dis_from(degp_ref[...])
    conv = dis * (sp_ref[0] + sp_ref[1] + tprev_ref[...]) + b_ref[...]
    hbn = _bn_relu(conv, g_ref[...], be_ref[...])
    out_ref[...] = dis * jnp.dot(hbn, wn_ref[...], preferred_element_type=_f32)


def _tc_mid(sp, tprev, degp, b, g, be, wn):
    return pl.pallas_call(
        _tc_mid_body,
        out_shape=jax.ShapeDtypeStruct((NP, DM), _f32),
    )(sp, tprev, degp, b.reshape(1, DM), g.reshape(1, DM), be.reshape(1, DM),
      wn)


def _tc_last_body(sp_ref, tprev_ref, degp_ref, b_ref, g_ref, be_ref, out_ref):
    dis = _dis_from(degp_ref[...])
    conv = dis * (sp_ref[0] + sp_ref[1] + tprev_ref[...]) + b_ref[...]
    out_ref[...] = _bn_relu(conv, g_ref[...], be_ref[...])


def _tc_last(sp, tprev, degp, b, g, be):
    return pl.pallas_call(
        _tc_last_body,
        out_shape=jax.ShapeDtypeStruct((NP, DM), _f32),
    )(sp, tprev, degp, b.reshape(1, DM), g.reshape(1, DM), be.reshape(1, DM))


_BE = 2048


def _tc_mlp_body(r_ref, c_ref, wa_ref, wb_ref, bm1_ref, wm2_ref, bm2_ref,
                 o_ref):
    r = r_ref[...]
    c = c_ref[...]
    mn = jnp.minimum(r, c)
    mx = jnp.maximum(r, c)
    z = (jnp.dot(mn, wa_ref[...], preferred_element_type=_f32)
         + jnp.dot(mx, wb_ref[...], preferred_element_type=_f32)
         + bm1_ref[...])
    o_ref[...] = (jnp.dot(jax.nn.relu(z), wm2_ref[...],
                          preferred_element_type=_f32) + bm2_ref[...])


def _tc_mlp(r, c, wm1, bm1, wm2, bm2):
    grid = (EP // _BE,)
    full = lambda s: pl.BlockSpec(s, lambda i: (0, 0))
    return pl.pallas_call(
        _tc_mlp_body,
        grid=grid,
        in_specs=[
            pl.BlockSpec((_BE, DM), lambda i: (i, 0)),
            pl.BlockSpec((_BE, DM), lambda i: (i, 0)),
            full((DM, 64)), full((DM, 64)), full((1, 64)),
            full((64, 1)), full((1, 1)),
        ],
        out_specs=pl.BlockSpec((_BE, 1), lambda i: (i, 0)),
        out_shape=jax.ShapeDtypeStruct((EP, 1), _f32),
    )(r, c, wm1[:DM], wm1[DM:], bm1.reshape(1, 64), wm2, bm2.reshape(1, 1))


def kernel(x, edge_index, batch, W0, b0, g0, be0, W1, b1, g1, be1,
           W2, b2, g2, be2, Wm1, bm1, Wm2, bm2):
    src = edge_index[0]
    dst = edge_index[1]
    pad = EP - EE
    padidx = NN + (jnp.arange(pad, dtype=jnp.int32) % 16)
    srcm = jnp.concatenate([src, padidx]).reshape(NW, NCHUNK, CH)
    dstm = jnp.concatenate([dst, padidx]).reshape(NW, NCHUNK, CH)

    zeros20 = jnp.zeros((NP, DM), _f32)
    zeros8 = jnp.zeros((NP, 8), _f32)
    ones8 = jnp.ones((CH, 8), _f32)
    xp = jnp.pad(x, ((0, NP - NN), (0, 0)))

    degp = _sc_deg(dstm, ones8, zeros8)
    t0 = _tc_a(xp, W0, degp)
    sp = _sc_scatter(t0, srcm, dstm, zeros20)
    t1 = _tc_mid(sp, t0, degp, b0, g0, be0, W1)
    sp = _sc_scatter(t1, srcm, dstm, zeros20)
    t2 = _tc_mid(sp, t1, degp, b1, g1, be1, W2)
    sp = _sc_scatter(t2, srcm, dstm, zeros20)
    h3 = _tc_last(sp, t2, degp, b2, g2, be2)
    r, c = _sc_edge(h3, srcm, dstm)
    out = _tc_mlp(r, c, Wm1, bm1, Wm2, bm2)
    return out[:EE]


# speedup vs baseline: 10.5136x; 1.0003x over previous
import functools

import jax
import jax.numpy as jnp
from jax import lax
from jax.experimental import pallas as pl
from jax.experimental.pallas import tpu as pltpu
from jax.experimental.pallas import tpu_sc as plsc

NN = 10000
EE = 320000
FI = 128
DM = 20
NP = 10112
DSC = 32
EP = 327680
NW = 32
CH = 128
NCHUNK = EP // NW // CH
RPS = NP // 16

_mesh = plsc.VectorSubcoreMesh(core_axis_name="c", subcore_axis_name="s")
_f32 = jnp.float32
_sc_params = pltpu.CompilerParams(use_tc_tiling_on_sc=False)


@functools.partial(
    pl.kernel, mesh=_mesh, compiler_params=_sc_params,
    out_type=jax.ShapeDtypeStruct((2, NP, 8), _f32),
    scratch_types=[
        pltpu.VMEM((NCHUNK, CH), jnp.int32),
        pltpu.VMEM((CH, 8), _f32),
        pltpu.VMEM_SHARED((NP, 8), _f32),
    ],
)
def _sc_deg(dstm, ones_hbm, zeros_hbm, out, didx, onesv, acc):
    cid = lax.axis_index("c")
    sid = lax.axis_index("s")
    wid = sid * 2 + cid
    pltpu.sync_copy(dstm.at[wid], didx)
    pltpu.sync_copy(ones_hbm, onesv)
    pltpu.sync_copy(zeros_hbm.at[pl.ds(sid * RPS, RPS)],
                    acc.at[pl.ds(sid * RPS, RPS)])
    plsc.subcore_barrier()

    def body(j, carry):
        pltpu.sync_copy(onesv, acc.at[didx.at[j]], add=True)
        return carry

    lax.fori_loop(0, NCHUNK, body, 0)
    plsc.subcore_barrier()
    pltpu.sync_copy(acc.at[pl.ds(sid * RPS, RPS)],
                    out.at[cid, pl.ds(sid * RPS, RPS)])


@functools.partial(
    pl.kernel, mesh=_mesh, compiler_params=_sc_params,
    out_type=jax.ShapeDtypeStruct((2, NP, DSC), _f32),
    scratch_types=[
        pltpu.VMEM((NCHUNK, CH), jnp.int32),
        pltpu.VMEM((NCHUNK, CH), jnp.int32),
        pltpu.VMEM((CH, DSC), _f32),
        pltpu.VMEM_SHARED((NP, DSC), _f32),
        pltpu.SemaphoreType.DMA,
    ],
)
def _sc_scatter(t_hbm, srcm, dstm, zeros_hbm, out, sidx, didx, rbuf0,
                acc, sem0):
    cid = lax.axis_index("c")
    sid = lax.axis_index("s")
    wid = sid * 2 + cid
    pltpu.sync_copy(srcm.at[wid], sidx)
    pltpu.sync_copy(dstm.at[wid], didx)
    pltpu.sync_copy(zeros_hbm.at[pl.ds(sid * RPS, RPS)],
                    acc.at[pl.ds(sid * RPS, RPS)])
    plsc.subcore_barrier()

    def body(j, carry):
        pltpu.async_copy(t_hbm.at[sidx.at[j]], rbuf0, sem0).wait()
        pltpu.sync_copy(rbuf0, acc.at[didx.at[j]], add=True)
        return carry

    lax.fori_loop(0, NCHUNK, body, 0)
    plsc.subcore_barrier()
    pltpu.sync_copy(acc.at[pl.ds(sid * RPS, RPS)],
                    out.at[cid, pl.ds(sid * RPS, RPS)])


@functools.partial(
    pl.kernel, mesh=_mesh, compiler_params=_sc_params,
    out_type=(jax.ShapeDtypeStruct((EP, DSC), _f32),
              jax.ShapeDtypeStruct((EP, DSC), _f32)),
    scratch_types=[
        pltpu.VMEM((NCHUNK, CH), jnp.int32),
        pltpu.VMEM((NCHUNK, CH), jnp.int32),
        pltpu.VMEM((CH, DSC), _f32),
        pltpu.VMEM((CH, DSC), _f32),
        pltpu.SemaphoreType.DMA,
        pltpu.SemaphoreType.DMA,
    ],
)
def _sc_edge(h_hbm, srcm, dstm, outr, outc, sidx, didx, rbuf0, cbuf0,
             sr0, sc0):
    cid = lax.axis_index("c")
    sid = lax.axis_index("s")
    wid = sid * 2 + cid
    pltpu.sync_copy(srcm.at[wid], sidx)
    pltpu.sync_copy(dstm.at[wid], didx)

    def body(j, carry):
        base = wid * (NCHUNK * CH) + j * CH
        pltpu.async_copy(h_hbm.at[sidx.at[j]], rbuf0, sr0).wait()
        pltpu.sync_copy(rbuf0, outr.at[pl.ds(base, CH)])
        pltpu.async_copy(h_hbm.at[didx.at[j]], cbuf0, sc0).wait()
        pltpu.sync_copy(cbuf0, outc.at[pl.ds(base, CH)])
        return carry

    lax.fori_loop(0, NCHUNK, body, 0)


def _dis_from(degp):
    deg = degp[0, :, 0:1] + degp[1, :, 0:1] + 1.0
    return 1.0 / jnp.sqrt(deg)


def _tc_a_body(x_ref, w0_ref, degp_ref, t0_ref):
    dis = _dis_from(degp_ref[...])
    h = jnp.dot(x_ref[...], w0_ref[...], preferred_element_type=_f32)
    t0_ref[...] = dis * h


def _tc_a(xp, w0, degp):
    return pl.pallas_call(
        _tc_a_body,
        out_shape=jax.ShapeDtypeStruct((NP, DSC), _f32),
    )(xp, w0, degp)


def _bn_relu(conv, g, be):
    mask = lax.broadcasted_iota(jnp.int32, (NP, 1), 0) < NN
    cm = jnp.where(mask, conv, 0.0)
    m = jnp.sum(cm, axis=0, keepdims=True) / NN
    d = conv - m
    v = jnp.sum(jnp.where(mask, d * d, 0.0), axis=0, keepdims=True) / NN
    return jax.nn.relu(g * d / jnp.sqrt(v + 1e-5) + be)


def _tc_mid_body(sp_ref, tprev_ref, degp_ref, b_ref, g_ref, be_ref, wn_ref,
                 out_ref):
    dis = _dis_from(degp_ref[...])
    conv = dis * (sp_ref[0] + sp_ref[1] + tprev_ref[...]) + b_ref[...]
    hbn = _bn_relu(conv, g_ref[...], be_ref[...])
    out_ref[...] = dis * jnp.dot(hbn, wn_ref[...], preferred_element_type=_f32)


def _tc_mid(sp, tprev, degp, b, g, be, wn):
    return pl.pallas_call(
        _tc_mid_body,
        out_shape=jax.ShapeDtypeStruct((NP, DSC), _f32),
    )(sp, tprev, degp, b.reshape(1, DSC), g.reshape(1, DSC),
      be.reshape(1, DSC), wn)


def _tc_last_body(sp_ref, tprev_ref, degp_ref, b_ref, g_ref, be_ref, out_ref):
    dis = _dis_from(degp_ref[...])
    conv = dis * (sp_ref[0] + sp_ref[1] + tprev_ref[...]) + b_ref[...]
    out_ref[...] = _bn_relu(conv, g_ref[...], be_ref[...])


def _tc_last(sp, tprev, degp, b, g, be):
    return pl.pallas_call(
        _tc_last_body,
        out_shape=jax.ShapeDtypeStruct((NP, DSC), _f32),
    )(sp, tprev, degp, b.reshape(1, DSC), g.reshape(1, DSC),
      be.reshape(1, DSC))


_BE = 2048

def _padr(w):
    return jnp.pad(w, ((0, DSC - DM), (0, 0)))


def _tc_mlp_body(r_ref, c_ref, wa_ref, wb_ref, bm1_ref, wm2_ref, bm2_ref,
                 o_ref):
    r = r_ref[...]
    c = c_ref[...]
    mn = jnp.minimum(r, c)
    mx = jnp.maximum(r, c)
    z = (jnp.dot(mn, wa_ref[...], preferred_element_type=_f32)
         + jnp.dot(mx, wb_ref[...], preferred_element_type=_f32)
         + bm1_ref[...])
    o_ref[...] = (jnp.dot(jax.nn.relu(z), wm2_ref[...],
                          preferred_element_type=_f32) + bm2_ref[...])


def _tc_mlp(r, c, wm1, bm1, wm2, bm2):
    grid = (EP // _BE,)
    full = lambda s: pl.BlockSpec(s, lambda i: (0, 0))
    return pl.pallas_call(
        _tc_mlp_body,
        grid=grid,
        in_specs=[
            pl.BlockSpec((_BE, DSC), lambda i: (i, 0)),
            pl.BlockSpec((_BE, DSC), lambda i: (i, 0)),
            full((DSC, 64)), full((DSC, 64)), full((1, 64)),
            full((64, 1)), full((1, 1)),
        ],
        out_specs=pl.BlockSpec((_BE, 1), lambda i: (i, 0)),
        out_shape=jax.ShapeDtypeStruct((EP, 1), _f32),
    )(r, c, _padr(wm1[:DM]), _padr(wm1[DM:]), bm1.reshape(1, 64), wm2,
      bm2.reshape(1, 1))


def kernel(x, edge_index, batch, W0, b0, g0, be0, W1, b1, g1, be1,
           W2, b2, g2, be2, Wm1, bm1, Wm2, bm2):
    src = edge_index[0]
    dst = edge_index[1]
    pad = EP - EE
    padidx = NN + (jnp.arange(pad, dtype=jnp.int32) % 16)
    srcm = jnp.concatenate([src, padidx]).reshape(NW, NCHUNK, CH)
    dstm = jnp.concatenate([dst, padidx]).reshape(NW, NCHUNK, CH)

    zeros32 = jnp.zeros((NP, DSC), _f32)
    zeros8 = jnp.zeros((NP, 8), _f32)
    ones8 = jnp.ones((CH, 8), _f32)
    xp = jnp.pad(x, ((0, NP - NN), (0, 0)))
    w0p = jnp.pad(W0, ((0, 0), (0, DSC - DM)))
    w1p = jnp.pad(W1, ((0, DSC - DM), (0, DSC - DM)))
    w2p = jnp.pad(W2, ((0, DSC - DM), (0, DSC - DM)))
    pv = lambda v: jnp.pad(v, (0, DSC - DM))

    degp = _sc_deg(dstm, ones8, zeros8)
    t0 = _tc_a(xp, w0p, degp)
    sp = _sc_scatter(t0, srcm, dstm, zeros32)
    t1 = _tc_mid(sp, t0, degp, pv(b0), pv(g0), pv(be0), w1p)
    sp = _sc_scatter(t1, srcm, dstm, zeros32)
    t2 = _tc_mid(sp, t1, degp, pv(b1), pv(g1), pv(be1), w2p)
    sp = _sc_scatter(t2, srcm, dstm, zeros32)
    h3 = _tc_last(sp, t2, degp, pv(b2), pv(g2), pv(be2))
    r, c = _sc_edge(h3, srcm, dstm)
    out = _tc_mlp(r, c, Wm1, bm1, Wm2, bm2)
    return out[:EE]


# speedup vs baseline: 12.1887x; 1.1593x over previous
import functools

import jax
import jax.numpy as jnp
from jax import lax
from jax.experimental import pallas as pl
from jax.experimental.pallas import tpu as pltpu
from jax.experimental.pallas import tpu_sc as plsc

NN = 10000
EE = 320000
FI = 128
DM = 20
NP = 10112
DSC = 32
EP = 327680
NW = 32
CH = 128
NCHUNK = EP // NW // CH
RPS = NP // 16

_mesh = plsc.VectorSubcoreMesh(core_axis_name="c", subcore_axis_name="s")
_f32 = jnp.float32
_sc_params = pltpu.CompilerParams(use_tc_tiling_on_sc=False)


@functools.partial(
    pl.kernel, mesh=_mesh, compiler_params=_sc_params,
    out_type=jax.ShapeDtypeStruct((2, NP, 8), _f32),
    scratch_types=[
        pltpu.VMEM((NCHUNK, CH), jnp.int32),
        pltpu.VMEM((CH, 8), _f32),
        pltpu.VMEM_SHARED((NP, 8), _f32),
    ],
)
def _sc_deg(dstm, ones_hbm, zeros_hbm, out, didx, onesv, acc):
    cid = lax.axis_index("c")
    sid = lax.axis_index("s")
    wid = sid * 2 + cid
    pltpu.sync_copy(dstm.at[wid], didx)
    pltpu.sync_copy(ones_hbm, onesv)
    pltpu.sync_copy(zeros_hbm.at[pl.ds(sid * RPS, RPS)],
                    acc.at[pl.ds(sid * RPS, RPS)])
    plsc.subcore_barrier()

    def body(j, carry):
        pltpu.sync_copy(onesv, acc.at[didx.at[j]], add=True)
        return carry

    lax.fori_loop(0, NCHUNK, body, 0)
    plsc.subcore_barrier()
    pltpu.sync_copy(acc.at[pl.ds(sid * RPS, RPS)],
                    out.at[cid, pl.ds(sid * RPS, RPS)])


@functools.partial(
    pl.kernel, mesh=_mesh, compiler_params=_sc_params,
    out_type=jax.ShapeDtypeStruct((2, NP, DSC), _f32),
    scratch_types=[
        pltpu.VMEM((NCHUNK, CH), jnp.int32),
        pltpu.VMEM((NCHUNK, CH), jnp.int32),
        pltpu.VMEM((CH, DSC), _f32),
        pltpu.VMEM((CH, DSC), _f32),
        pltpu.VMEM_SHARED((NP, DSC), _f32),
        pltpu.SemaphoreType.DMA,
        pltpu.SemaphoreType.DMA,
    ],
)
def _sc_scatter(t_hbm, srcm, dstm, zeros_hbm, out, sidx, didx, rbuf0, rbuf1,
                acc, sem0, sem1):
    cid = lax.axis_index("c")
    sid = lax.axis_index("s")
    wid = sid * 2 + cid
    pltpu.sync_copy(srcm.at[wid], sidx)
    pltpu.sync_copy(dstm.at[wid], didx)
    pltpu.sync_copy(zeros_hbm.at[pl.ds(sid * RPS, RPS)],
                    acc.at[pl.ds(sid * RPS, RPS)])
    plsc.subcore_barrier()

    pltpu.async_copy(t_hbm.at[sidx.at[0]], rbuf0, sem0)

    def body(jj, carry):
        j0 = 2 * jj
        pltpu.async_copy(t_hbm.at[sidx.at[j0 + 1]], rbuf1, sem1)
        pltpu.make_async_copy(t_hbm.at[sidx.at[j0]], rbuf0, sem0).wait()
        pltpu.sync_copy(rbuf0, acc.at[didx.at[j0]], add=True)

        @pl.when(jj < NCHUNK // 2 - 1)
        def _():
            pltpu.async_copy(t_hbm.at[sidx.at[j0 + 2]], rbuf0, sem0)

        pltpu.make_async_copy(t_hbm.at[sidx.at[j0 + 1]], rbuf1, sem1).wait()
        pltpu.sync_copy(rbuf1, acc.at[didx.at[j0 + 1]], add=True)
        return carry

    lax.fori_loop(0, NCHUNK // 2, body, 0)
    plsc.subcore_barrier()
    pltpu.sync_copy(acc.at[pl.ds(sid * RPS, RPS)],
                    out.at[cid, pl.ds(sid * RPS, RPS)])


@functools.partial(
    pl.kernel, mesh=_mesh, compiler_params=_sc_params,
    out_type=(jax.ShapeDtypeStruct((EP, DSC), _f32),
              jax.ShapeDtypeStruct((EP, DSC), _f32)),
    scratch_types=[
        pltpu.VMEM((NCHUNK, CH), jnp.int32),
        pltpu.VMEM((NCHUNK, CH), jnp.int32),
        pltpu.VMEM((CH, DSC), _f32),
        pltpu.VMEM((CH, DSC), _f32),
        pltpu.SemaphoreType.DMA,
        pltpu.SemaphoreType.DMA,
    ],
)
def _sc_edge(h_hbm, srcm, dstm, outr, outc, sidx, didx, rbuf0, cbuf0,
             sr0, sc0):
    cid = lax.axis_index("c")
    sid = lax.axis_index("s")
    wid = sid * 2 + cid
    pltpu.sync_copy(srcm.at[wid], sidx)
    pltpu.sync_copy(dstm.at[wid], didx)

    def body(j, carry):
        base = wid * (NCHUNK * CH) + j * CH
        pltpu.async_copy(h_hbm.at[sidx.at[j]], rbuf0, sr0)
        pltpu.async_copy(h_hbm.at[didx.at[j]], cbuf0, sc0)
        pltpu.make_async_copy(h_hbm.at[sidx.at[j]], rbuf0, sr0).wait()
        pltpu.sync_copy(rbuf0, outr.at[pl.ds(base, CH)])
        pltpu.make_async_copy(h_hbm.at[didx.at[j]], cbuf0, sc0).wait()
        pltpu.sync_copy(cbuf0, outc.at[pl.ds(base, CH)])
        return carry

    lax.fori_loop(0, NCHUNK, body, 0)


def _dis_from(degp):
    deg = degp[0, :, 0:1] + degp[1, :, 0:1] + 1.0
    return 1.0 / jnp.sqrt(deg)


def _tc_a_body(x_ref, w0_ref, degp_ref, t0_ref):
    dis = _dis_from(degp_ref[...])
    h = jnp.dot(x_ref[...], w0_ref[...], preferred_element_type=_f32)
    t0_ref[...] = dis * h


def _tc_a(xp, w0, degp):
    return pl.pallas_call(
        _tc_a_body,
        out_shape=jax.ShapeDtypeStruct((NP, DSC), _f32),
    )(xp, w0, degp)


def _bn_relu(conv, g, be):
    mask = lax.broadcasted_iota(jnp.int32, (NP, 1), 0) < NN
    cm = jnp.where(mask, conv, 0.0)
    m = jnp.sum(cm, axis=0, keepdims=True) / NN
    d = conv - m
    v = jnp.sum(jnp.where(mask, d * d, 0.0), axis=0, keepdims=True) / NN
    return jax.nn.relu(g * d / jnp.sqrt(v + 1e-5) + be)


def _tc_mid_body(sp_ref, tprev_ref, degp_ref, b_ref, g_ref, be_ref, wn_ref,
                 out_ref):
    dis = _dis_from(degp_ref[...])
    conv = dis * (sp_ref[0] + sp_ref[1] + tprev_ref[...]) + b_ref[...]
    hbn = _bn_relu(conv, g_ref[...], be_ref[...])
    out_ref[...] = dis * jnp.dot(hbn, wn_ref[...], preferred_element_type=_f32)


def _tc_mid(sp, tprev, degp, b, g, be, wn):
    return pl.pallas_call(
        _tc_mid_body,
        out_shape=jax.ShapeDtypeStruct((NP, DSC), _f32),
    )(sp, tprev, degp, b.reshape(1, DSC), g.reshape(1, DSC),
      be.reshape(1, DSC), wn)


def _tc_last_body(sp_ref, tprev_ref, degp_ref, b_ref, g_ref, be_ref, out_ref):
    dis = _dis_from(degp_ref[...])
    conv = dis * (sp_ref[0] + sp_ref[1] + tprev_ref[...]) + b_ref[...]
    out_ref[...] = _bn_relu(conv, g_ref[...], be_ref[...])


def _tc_last(sp, tprev, degp, b, g, be):
    return pl.pallas_call(
        _tc_last_body,
        out_shape=jax.ShapeDtypeStruct((NP, DSC), _f32),
    )(sp, tprev, degp, b.reshape(1, DSC), g.reshape(1, DSC),
      be.reshape(1, DSC))


_BE = 2048

def _padr(w):
    return jnp.pad(w, ((0, DSC - DM), (0, 0)))


def _tc_mlp_body(r_ref, c_ref, wa_ref, wb_ref, bm1_ref, wm2_ref, bm2_ref,
                 o_ref):
    r = r_ref[...]
    c = c_ref[...]
    mn = jnp.minimum(r, c)
    mx = jnp.maximum(r, c)
    z = (jnp.dot(mn, wa_ref[...], preferred_element_type=_f32)
         + jnp.dot(mx, wb_ref[...], preferred_element_type=_f32)
         + bm1_ref[...])
    o_ref[...] = (jnp.dot(jax.nn.relu(z), wm2_ref[...],
                          preferred_element_type=_f32) + bm2_ref[...])


def _tc_mlp(r, c, wm1, bm1, wm2, bm2):
    grid = (EP // _BE,)
    full = lambda s: pl.BlockSpec(s, lambda i: (0, 0))
    return pl.pallas_call(
        _tc_mlp_body,
        grid=grid,
        in_specs=[
            pl.BlockSpec((_BE, DSC), lambda i: (i, 0)),
            pl.BlockSpec((_BE, DSC), lambda i: (i, 0)),
            full((DSC, 64)), full((DSC, 64)), full((1, 64)),
            full((64, 1)), full((1, 1)),
        ],
        out_specs=pl.BlockSpec((_BE, 1), lambda i: (i, 0)),
        out_shape=jax.ShapeDtypeStruct((EP, 1), _f32),
    )(r, c, _padr(wm1[:DM]), _padr(wm1[DM:]), bm1.reshape(1, 64), wm2,
      bm2.reshape(1, 1))


def kernel(x, edge_index, batch, W0, b0, g0, be0, W1, b1, g1, be1,
           W2, b2, g2, be2, Wm1, bm1, Wm2, bm2):
    src = edge_index[0]
    dst = edge_index[1]
    pad = EP - EE
    padidx = NN + (jnp.arange(pad, dtype=jnp.int32) % 16)
    srcm = jnp.concatenate([src, padidx]).reshape(NW, NCHUNK, CH)
    dstm = jnp.concatenate([dst, padidx]).reshape(NW, NCHUNK, CH)

    zeros32 = jnp.zeros((NP, DSC), _f32)
    zeros8 = jnp.zeros((NP, 8), _f32)
    ones8 = jnp.ones((CH, 8), _f32)
    xp = jnp.pad(x, ((0, NP - NN), (0, 0)))
    w0p = jnp.pad(W0, ((0, 0), (0, DSC - DM)))
    w1p = jnp.pad(W1, ((0, DSC - DM), (0, DSC - DM)))
    w2p = jnp.pad(W2, ((0, DSC - DM), (0, DSC - DM)))
    pv = lambda v: jnp.pad(v, (0, DSC - DM))

    degp = _sc_deg(dstm, ones8, zeros8)
    t0 = _tc_a(xp, w0p, degp)
    sp = _sc_scatter(t0, srcm, dstm, zeros32)
    t1 = _tc_mid(sp, t0, degp, pv(b0), pv(g0), pv(be0), w1p)
    sp = _sc_scatter(t1, srcm, dstm, zeros32)
    t2 = _tc_mid(sp, t1, degp, pv(b1), pv(g1), pv(be1), w2p)
    sp = _sc_scatter(t2, srcm, dstm, zeros32)
    h3 = _tc_last(sp, t2, degp, pv(b2), pv(g2), pv(be2))
    r, c = _sc_edge(h3, srcm, dstm)
    out = _tc_mlp(r, c, Wm1, bm1, Wm2, bm2)
    return out[:EE]


# speedup vs baseline: 14.5687x; 1.1953x over previous
import functools

import jax
import jax.numpy as jnp
from jax import lax
from jax.experimental import pallas as pl
from jax.experimental.pallas import tpu as pltpu
from jax.experimental.pallas import tpu_sc as plsc

NN = 10000
EE = 320000
FI = 128
DM = 20
NP = 10112
DSC = 32
EP = 327680
NW = 32
CH = 128
NCHUNK = EP // NW // CH
RPS = NP // 16

_mesh = plsc.VectorSubcoreMesh(core_axis_name="c", subcore_axis_name="s")
_f32 = jnp.float32
_sc_params = pltpu.CompilerParams(use_tc_tiling_on_sc=False)


@functools.partial(
    pl.kernel, mesh=_mesh, compiler_params=_sc_params,
    out_type=jax.ShapeDtypeStruct((2, NP, 8), _f32),
    scratch_types=[
        pltpu.VMEM((NCHUNK, CH), jnp.int32),
        pltpu.VMEM((CH, 8), _f32),
        pltpu.VMEM_SHARED((NP, 8), _f32),
    ],
)
def _sc_deg(dstm, ones_hbm, zeros_hbm, out, didx, onesv, acc):
    cid = lax.axis_index("c")
    sid = lax.axis_index("s")
    wid = sid * 2 + cid
    pltpu.sync_copy(dstm.at[wid], didx)
    pltpu.sync_copy(ones_hbm, onesv)
    pltpu.sync_copy(zeros_hbm.at[pl.ds(sid * RPS, RPS)],
                    acc.at[pl.ds(sid * RPS, RPS)])
    plsc.subcore_barrier()

    def body(j, carry):
        pltpu.sync_copy(onesv, acc.at[didx.at[j]], add=True)
        return carry

    lax.fori_loop(0, NCHUNK, body, 0)
    plsc.subcore_barrier()
    pltpu.sync_copy(acc.at[pl.ds(sid * RPS, RPS)],
                    out.at[cid, pl.ds(sid * RPS, RPS)])


@functools.partial(
    pl.kernel, mesh=_mesh, compiler_params=_sc_params,
    out_type=jax.ShapeDtypeStruct((2, NP, DSC), _f32),
    scratch_types=[
        pltpu.VMEM((NCHUNK, CH), jnp.int32),
        pltpu.VMEM((NCHUNK, CH), jnp.int32),
        pltpu.VMEM((CH, DSC), _f32),
        pltpu.VMEM((CH, DSC), _f32),
        pltpu.VMEM_SHARED((NP, DSC), _f32),
        pltpu.VMEM_SHARED((NP, DSC), _f32),
        pltpu.SemaphoreType.DMA,
        pltpu.SemaphoreType.DMA,
    ],
)
def _sc_scatter(t_hbm, srcm, dstm, zeros_hbm, out, sidx, didx, rbuf0, rbuf1,
                acc, tsp, sem0, sem1):
    cid = lax.axis_index("c")
    sid = lax.axis_index("s")
    wid = sid * 2 + cid
    pltpu.sync_copy(srcm.at[wid], sidx)
    pltpu.sync_copy(dstm.at[wid], didx)
    pltpu.sync_copy(zeros_hbm.at[pl.ds(sid * RPS, RPS)],
                    acc.at[pl.ds(sid * RPS, RPS)])
    pltpu.sync_copy(t_hbm.at[pl.ds(sid * RPS, RPS)],
                    tsp.at[pl.ds(sid * RPS, RPS)])
    plsc.subcore_barrier()

    pltpu.async_copy(tsp.at[sidx.at[0]], rbuf0, sem0)

    def body(jj, carry):
        j0 = 2 * jj
        pltpu.async_copy(tsp.at[sidx.at[j0 + 1]], rbuf1, sem1)
        pltpu.make_async_copy(tsp.at[sidx.at[j0]], rbuf0, sem0).wait()
        pltpu.sync_copy(rbuf0, acc.at[didx.at[j0]], add=True)

        @pl.when(jj < NCHUNK // 2 - 1)
        def _():
            pltpu.async_copy(tsp.at[sidx.at[j0 + 2]], rbuf0, sem0)

        pltpu.make_async_copy(tsp.at[sidx.at[j0 + 1]], rbuf1, sem1).wait()
        pltpu.sync_copy(rbuf1, acc.at[didx.at[j0 + 1]], add=True)
        return carry

    lax.fori_loop(0, NCHUNK // 2, body, 0)
    plsc.subcore_barrier()
    pltpu.sync_copy(acc.at[pl.ds(sid * RPS, RPS)],
                    out.at[cid, pl.ds(sid * RPS, RPS)])


@functools.partial(
    pl.kernel, mesh=_mesh, compiler_params=_sc_params,
    out_type=(jax.ShapeDtypeStruct((EP, DSC), _f32),
              jax.ShapeDtypeStruct((EP, DSC), _f32)),
    scratch_types=[
        pltpu.VMEM((NCHUNK, CH), jnp.int32),
        pltpu.VMEM((NCHUNK, CH), jnp.int32),
        pltpu.VMEM((CH, DSC), _f32),
        pltpu.VMEM((CH, DSC), _f32),
        pltpu.VMEM((CH, DSC), _f32),
        pltpu.VMEM((CH, DSC), _f32),
        pltpu.VMEM_SHARED((NP, DSC), _f32),
        pltpu.SemaphoreType.DMA,
        pltpu.SemaphoreType.DMA,
        pltpu.SemaphoreType.DMA,
        pltpu.SemaphoreType.DMA,
    ],
)
def _sc_edge(h_hbm, srcm, dstm, outr, outc, sidx, didx, rbuf0, rbuf1,
             cbuf0, cbuf1, hsp, sr0, sr1, sc0, sc1):
    cid = lax.axis_index("c")
    sid = lax.axis_index("s")
    wid = sid * 2 + cid
    pltpu.sync_copy(srcm.at[wid], sidx)
    pltpu.sync_copy(dstm.at[wid], didx)
    pltpu.sync_copy(h_hbm.at[pl.ds(sid * RPS, RPS)],
                    hsp.at[pl.ds(sid * RPS, RPS)])
    plsc.subcore_barrier()

    pltpu.async_copy(hsp.at[sidx.at[0]], rbuf0, sr0)
    pltpu.async_copy(hsp.at[didx.at[0]], cbuf0, sc0)

    def body(jj, carry):
        j0 = 2 * jj
        base = wid * (NCHUNK * CH) + j0 * CH
        pltpu.async_copy(hsp.at[sidx.at[j0 + 1]], rbuf1, sr1)
        pltpu.async_copy(hsp.at[didx.at[j0 + 1]], cbuf1, sc1)
        pltpu.make_async_copy(hsp.at[sidx.at[j0]], rbuf0, sr0).wait()
        pltpu.sync_copy(rbuf0, outr.at[pl.ds(base, CH)])
        pltpu.make_async_copy(hsp.at[didx.at[j0]], cbuf0, sc0).wait()
        pltpu.sync_copy(cbuf0, outc.at[pl.ds(base, CH)])

        @pl.when(jj < NCHUNK // 2 - 1)
        def _():
            pltpu.async_copy(hsp.at[sidx.at[j0 + 2]], rbuf0, sr0)
            pltpu.async_copy(hsp.at[didx.at[j0 + 2]], cbuf0, sc0)

        pltpu.make_async_copy(hsp.at[sidx.at[j0 + 1]], rbuf1, sr1).wait()
        pltpu.sync_copy(rbuf1, outr.at[pl.ds(base + CH, CH)])
        pltpu.make_async_copy(hsp.at[didx.at[j0 + 1]], cbuf1, sc1).wait()
        pltpu.sync_copy(cbuf1, outc.at[pl.ds(base + CH, CH)])
        return carry

    lax.fori_loop(0, NCHUNK // 2, body, 0)


def _dis_from(degp):
    deg = degp[0, :, 0:1] + degp[1, :, 0:1] + 1.0
    return 1.0 / jnp.sqrt(deg)


def _tc_a_body(x_ref, w0_ref, degp_ref, t0_ref):
    dis = _dis_from(degp_ref[...])
    h = jnp.dot(x_ref[...], w0_ref[...], preferred_element_type=_f32)
    t0_ref[...] = dis * h


def _tc_a(xp, w0, degp):
    return pl.pallas_call(
        _tc_a_body,
        out_shape=jax.ShapeDtypeStruct((NP, DSC), _f32),
    )(xp, w0, degp)


def _bn_relu(conv, g, be):
    mask = lax.broadcasted_iota(jnp.int32, (NP, 1), 0) < NN
    cm = jnp.where(mask, conv, 0.0)
    m = jnp.sum(cm, axis=0, keepdims=True) / NN
    d = conv - m
    v = jnp.sum(jnp.where(mask, d * d, 0.0), axis=0, keepdims=True) / NN
    return jax.nn.relu(g * d / jnp.sqrt(v + 1e-5) + be)


def _tc_mid_body(sp_ref, tprev_ref, degp_ref, b_ref, g_ref, be_ref, wn_ref,
                 out_ref):
    dis = _dis_from(degp_ref[...])
    conv = dis * (sp_ref[0] + sp_ref[1] + tprev_ref[...]) + b_ref[...]
    hbn = _bn_relu(conv, g_ref[...], be_ref[...])
    out_ref[...] = dis * jnp.dot(hbn, wn_ref[...], preferred_element_type=_f32)


def _tc_mid(sp, tprev, degp, b, g, be, wn):
    return pl.pallas_call(
        _tc_mid_body,
        out_shape=jax.ShapeDtypeStruct((NP, DSC), _f32),
    )(sp, tprev, degp, b.reshape(1, DSC), g.reshape(1, DSC),
      be.reshape(1, DSC), wn)


def _tc_last_body(sp_ref, tprev_ref, degp_ref, b_ref, g_ref, be_ref, out_ref):
    dis = _dis_from(degp_ref[...])
    conv = dis * (sp_ref[0] + sp_ref[1] + tprev_ref[...]) + b_ref[...]
    out_ref[...] = _bn_relu(conv, g_ref[...], be_ref[...])


def _tc_last(sp, tprev, degp, b, g, be):
    return pl.pallas_call(
        _tc_last_body,
        out_shape=jax.ShapeDtypeStruct((NP, DSC), _f32),
    )(sp, tprev, degp, b.reshape(1, DSC), g.reshape(1, DSC),
      be.reshape(1, DSC))


_BE = 2048

def _padr(w):
    return jnp.pad(w, ((0, DSC - DM), (0, 0)))


def _tc_mlp_body(r_ref, c_ref, wa_ref, wb_ref, bm1_ref, wm2_ref, bm2_ref,
                 o_ref):
    r = r_ref[...]
    c = c_ref[...]
    mn = jnp.minimum(r, c)
    mx = jnp.maximum(r, c)
    z = (jnp.dot(mn, wa_ref[...], preferred_element_type=_f32)
         + jnp.dot(mx, wb_ref[...], preferred_element_type=_f32)
         + bm1_ref[...])
    o_ref[...] = (jnp.dot(jax.nn.relu(z), wm2_ref[...],
                          preferred_element_type=_f32) + bm2_ref[...])


def _tc_mlp(r, c, wm1, bm1, wm2, bm2):
    grid = (EP // _BE,)
    full = lambda s: pl.BlockSpec(s, lambda i: (0, 0))
    return pl.pallas_call(
        _tc_mlp_body,
        grid=grid,
        in_specs=[
            pl.BlockSpec((_BE, DSC), lambda i: (i, 0)),
            pl.BlockSpec((_BE, DSC), lambda i: (i, 0)),
            full((DSC, 64)), full((DSC, 64)), full((1, 64)),
            full((64, 1)), full((1, 1)),
        ],
        out_specs=pl.BlockSpec((_BE, 1), lambda i: (i, 0)),
        out_shape=jax.ShapeDtypeStruct((EP, 1), _f32),
    )(r, c, _padr(wm1[:DM]), _padr(wm1[DM:]), bm1.reshape(1, 64), wm2,
      bm2.reshape(1, 1))


def kernel(x, edge_index, batch, W0, b0, g0, be0, W1, b1, g1, be1,
           W2, b2, g2, be2, Wm1, bm1, Wm2, bm2):
    src = edge_index[0]
    dst = edge_index[1]
    pad = EP - EE
    padidx = NN + (jnp.arange(pad, dtype=jnp.int32) % 16)
    srcm = jnp.concatenate([src, padidx]).reshape(NW, NCHUNK, CH)
    dstm = jnp.concatenate([dst, padidx]).reshape(NW, NCHUNK, CH)

    zeros32 = jnp.zeros((NP, DSC), _f32)
    zeros8 = jnp.zeros((NP, 8), _f32)
    ones8 = jnp.ones((CH, 8), _f32)
    xp = jnp.pad(x, ((0, NP - NN), (0, 0)))
    w0p = jnp.pad(W0, ((0, 0), (0, DSC - DM)))
    w1p = jnp.pad(W1, ((0, DSC - DM), (0, DSC - DM)))
    w2p = jnp.pad(W2, ((0, DSC - DM), (0, DSC - DM)))
    pv = lambda v: jnp.pad(v, (0, DSC - DM))

    degp = _sc_deg(dstm, ones8, zeros8)
    t0 = _tc_a(xp, w0p, degp)
    sp = _sc_scatter(t0, srcm, dstm, zeros32)
    t1 = _tc_mid(sp, t0, degp, pv(b0), pv(g0), pv(be0), w1p)
    sp = _sc_scatter(t1, srcm, dstm, zeros32)
    t2 = _tc_mid(sp, t1, degp, pv(b1), pv(g1), pv(be1), w2p)
    sp = _sc_scatter(t2, srcm, dstm, zeros32)
    h3 = _tc_last(sp, t2, degp, pv(b2), pv(g2), pv(be2))
    r, c = _sc_edge(h3, srcm, dstm)
    out = _tc_mlp(r, c, Wm1, bm1, Wm2, bm2)
    return out[:EE]
